# baseline (speedup 1.0000x reference)
import functools

import jax
import jax.numpy as jnp
from jax import lax
from jax.experimental import pallas as pl
from jax.experimental.pallas import tpu as pltpu
from jax.experimental.pallas import tpu_sc as plsc

N, E, D = 10000, 320000, 128
NC, NS, NW = 2, 16, 32
EPT = E // NW
QD = D // 4
CHUNK = 128
GROUP = 5
NGROUP = 32
E_PAD = NS * NGROUP * GROUP * CHUNK
P_R = 10112
ACC_R = 10112
STRIPE = ACC_R // NS

def _deg_body(dst_hbm, degp_hbm, dchunk, deg_v):
    c = lax.axis_index("c")
    s = lax.axis_index("s")
    wid = s * NC + c
    pltpu.sync_copy(dst_hbm.at[pl.ds(wid * EPT, EPT)], dchunk)

    def zero(i, carry):
        deg_v[pl.ds(i * 16, 16)] = jnp.zeros((16,), jnp.float32)
        return carry

    lax.fori_loop(0, ACC_R // 16, zero, 0)

    ones = jnp.ones((16,), jnp.float32)

    def count(i, carry):
        idx = dchunk[pl.ds(i * 16, 16)]
        plsc.addupdate_scatter(deg_v, [idx], ones)
        return carry

    lax.fori_loop(0, EPT // 16, count, 0)
    pltpu.sync_copy(deg_v, degp_hbm.at[wid])


def _scatter_body(p4_hbm, src_hbm, dst_hbm, zrows_hbm, acc_hbm,
                  sidx_v, didx_v, rows_v, acc_sh, p_sh, gsem):
    c = lax.axis_index("c")
    s = lax.axis_index("s")
    base = s * NGROUP
    row0 = s * STRIPE

    for t in range(2):
        qq = t * NC + c
        pltpu.sync_copy(p4_hbm.at[qq, pl.ds(row0, STRIPE)],
                        p_sh.at[pl.ds(row0, STRIPE)])
        pltpu.sync_copy(zrows_hbm, acc_sh.at[pl.ds(row0, STRIPE)])
        plsc.subcore_barrier()

        def group(g, carry):
            gg = base + g
            pltpu.sync_copy(src_hbm.at[gg], sidx_v)
            pltpu.sync_copy(dst_hbm.at[gg], didx_v)
            handles = [
                pltpu.async_copy(
                    p_sh.at[sidx_v.at[k]],
                    rows_v.at[pl.ds(k * CHUNK, CHUNK)],
                    gsem,
                )
                for k in range(GROUP)
            ]
            for h in handles:
                h.wait()
            for k in range(GROUP):
                pltpu.sync_copy(
                    rows_v.at[pl.ds(k * CHUNK, CHUNK)],
                    acc_sh.at[didx_v.at[k]],
                    add=True,
                )
            return carry

        lax.fori_loop(0, NGROUP, group, 0)
        plsc.subcore_barrier()
        pltpu.sync_copy(acc_sh.at[pl.ds(row0, STRIPE)],
                        acc_hbm.at[qq, pl.ds(row0, STRIPE)])


@functools.lru_cache(maxsize=None)
def _sc_kernels():
    mesh = plsc.VectorSubcoreMesh(core_axis_name="c", subcore_axis_name="s")
    deg = pl.kernel(
        _deg_body,
        mesh=mesh,
        out_type=jax.ShapeDtypeStruct((NW, ACC_R), jnp.float32),
        scratch_types=[
            pltpu.VMEM((EPT,), jnp.int32),
            pltpu.VMEM((ACC_R,), jnp.float32),
        ],
        compiler_params=pltpu.CompilerParams(needs_layout_passes=False),
    )
    scatter = pl.kernel(
        _scatter_body,
        mesh=mesh,
        out_type=jax.ShapeDtypeStruct((4, ACC_R, QD), jnp.float32),
        scratch_types=[
            pltpu.VMEM((GROUP, CHUNK), jnp.int32),
            pltpu.VMEM((GROUP, CHUNK), jnp.int32),
            pltpu.VMEM((GROUP * CHUNK, QD), jnp.float32),
            pltpu.VMEM_SHARED((ACC_R, QD), jnp.float32),
            pltpu.VMEM_SHARED((P_R, QD), jnp.float32),
            pltpu.SemaphoreType.DMA,
        ],
        compiler_params=pltpu.CompilerParams(use_tc_tiling_on_sc=False),
    )
    return deg, scatter


def _scale_body(degp_ref, x_ref, p4_ref, dinv_ref):
    ones = jnp.ones((NW, 1), jnp.float32)
    deg = lax.dot_general(
        degp_ref[...], ones, (((0,), (0,)), ((), ())),
        preferred_element_type=jnp.float32,
    )
    dinv = lax.rsqrt(deg + 1.0)
    dinv_ref[...] = dinv
    xd = x_ref[...] * dinv[:N]
    for q in range(4):
        p4_ref[q, :N] = xd[:, q * QD:(q + 1) * QD]
        p4_ref[q, N:] = jnp.zeros((P_R - N, QD), jnp.float32)


def _final_body(acc_ref, x_ref, dinv_ref, w_ref, b_ref, a_ref, o_ref):
    ssum = jnp.concatenate([acc_ref[q, :N] for q in range(4)], axis=1)
    dinv = dinv_ref[:N]
    q = (ssum + x_ref[...] * dinv) * dinv
    z = jnp.dot(q, w_ref[...], preferred_element_type=jnp.float32) + b_ref[...]
    o_ref[...] = jnp.where(z >= 0, z, a_ref[...] * z)


def kernel(x, edge_index, W, b, prelu_a):
    pad = jnp.full((E_PAD - E,), N, jnp.int32)
    src3d = jnp.concatenate([edge_index[0], pad]).reshape(-1, GROUP, CHUNK)
    dst3d = jnp.concatenate([edge_index[1], pad]).reshape(-1, GROUP, CHUNK)
    zrows = jnp.zeros((STRIPE, QD), jnp.float32)

    deg_kernel, scatter_kernel = _sc_kernels()
    degp = deg_kernel(edge_index[1])
    p4, dinv = pl.pallas_call(
        _scale_body,
        out_shape=(
            jax.ShapeDtypeStruct((4, P_R, QD), jnp.float32),
            jax.ShapeDtypeStruct((ACC_R, 1), jnp.float32),
        ),
    )(degp, x)
    acc4 = scatter_kernel(p4, src3d, dst3d, zrows)
    out = pl.pallas_call(
        _final_body,
        out_shape=jax.ShapeDtypeStruct((N, D), jnp.float32),
    )(acc4, x, dinv, W, b.reshape(1, D), prelu_a.reshape(1, 1))
    return out


# speedup vs baseline: 1.3382x; 1.3382x over previous
import functools

import jax
import jax.numpy as jnp
from jax import lax
from jax.experimental import pallas as pl
from jax.experimental.pallas import tpu as pltpu
from jax.experimental.pallas import tpu_sc as plsc

N, E, D = 10000, 320000, 128
NC, NS, NW = 2, 16, 32
EPT = E // NW
QD = D // 4
CHUNK = 128
GROUP = 5
NGROUP = 32
E_PAD = NS * NGROUP * GROUP * CHUNK
P_R = 10112
ACC_R = 10112
STRIPE = ACC_R // NS

def _deg_body(dst_hbm, degp_hbm, dchunk, deg_v):
    c = lax.axis_index("c")
    s = lax.axis_index("s")
    wid = s * NC + c
    pltpu.sync_copy(dst_hbm.at[pl.ds(wid * EPT, EPT)], dchunk)

    def zero(i, carry):
        deg_v[pl.ds(i * 16, 16)] = jnp.zeros((16,), jnp.float32)
        return carry

    lax.fori_loop(0, ACC_R // 16, zero, 0)

    ones = jnp.ones((16,), jnp.float32)

    def count(i, carry):
        idx = dchunk[pl.ds(i * 16, 16)]
        plsc.addupdate_scatter(deg_v, [idx], ones)
        return carry

    lax.fori_loop(0, EPT // 16, count, 0)
    pltpu.sync_copy(deg_v, degp_hbm.at[wid])


def _scatter_body(p4_hbm, src_hbm, dst_hbm, zrows_hbm, acc_hbm,
                  sidx_all, didx_all, rows_a, rows_b, acc_sh, p_sh,
                  gsem, ssem):
    c = lax.axis_index("c")
    s = lax.axis_index("s")
    row0 = s * STRIPE

    pltpu.sync_copy(src_hbm.at[s], sidx_all)
    pltpu.sync_copy(dst_hbm.at[s], didx_all)

    def fire_gathers(g, rows):
        for k in range(GROUP):
            pltpu.async_copy(p_sh.at[sidx_all.at[g * GROUP + k]],
                             rows.at[pl.ds(k * CHUNK, CHUNK)], gsem)

    def drain_gathers(g, rows):
        for k in range(GROUP):
            pltpu.make_async_copy(p_sh.at[sidx_all.at[g * GROUP + k]],
                                  rows.at[pl.ds(k * CHUNK, CHUNK)],
                                  gsem).wait()

    def scatters(g, rows):
        handles = [
            pltpu.async_copy(rows.at[pl.ds(k * CHUNK, CHUNK)],
                             acc_sh.at[didx_all.at[g * GROUP + k]],
                             ssem, add=True)
            for k in range(GROUP)
        ]
        for h in handles:
            h.wait()

    for t in range(2):
        qq = t * NC + c
        pltpu.sync_copy(p4_hbm.at[qq, pl.ds(row0, STRIPE)],
                        p_sh.at[pl.ds(row0, STRIPE)])
        pltpu.sync_copy(zrows_hbm, acc_sh.at[pl.ds(row0, STRIPE)])
        plsc.subcore_barrier()

        fire_gathers(0, rows_a)

        def body(t2, carry):
            g = t2 * 2
            fire_gathers(g + 1, rows_b)
            drain_gathers(g, rows_a)
            scatters(g, rows_a)

            @pl.when(t2 < NGROUP // 2 - 1)
            def _():
                fire_gathers(g + 2, rows_a)

            drain_gathers(g + 1, rows_b)
            scatters(g + 1, rows_b)
            return carry

        lax.fori_loop(0, NGROUP // 2, body, 0)
        plsc.subcore_barrier()
        pltpu.sync_copy(acc_sh.at[pl.ds(row0, STRIPE)],
                        acc_hbm.at[qq, pl.ds(row0, STRIPE)])


@functools.lru_cache(maxsize=None)
def _sc_kernels():
    mesh = plsc.VectorSubcoreMesh(core_axis_name="c", subcore_axis_name="s")
    deg = pl.kernel(
        _deg_body,
        mesh=mesh,
        out_type=jax.ShapeDtypeStruct((NW, ACC_R), jnp.float32),
        scratch_types=[
            pltpu.VMEM((EPT,), jnp.int32),
            pltpu.VMEM((ACC_R,), jnp.float32),
        ],
        compiler_params=pltpu.CompilerParams(needs_layout_passes=False),
    )
    scatter = pl.kernel(
        _scatter_body,
        mesh=mesh,
        out_type=jax.ShapeDtypeStruct((4, ACC_R, QD), jnp.float32),
        scratch_types=[
            pltpu.VMEM((NGROUP * GROUP, CHUNK), jnp.int32),
            pltpu.VMEM((NGROUP * GROUP, CHUNK), jnp.int32),
            pltpu.VMEM((GROUP * CHUNK, QD), jnp.float32),
            pltpu.VMEM((GROUP * CHUNK, QD), jnp.float32),
            pltpu.VMEM_SHARED((ACC_R, QD), jnp.float32),
            pltpu.VMEM_SHARED((P_R, QD), jnp.float32),
            pltpu.SemaphoreType.DMA,
            pltpu.SemaphoreType.DMA,
        ],
        compiler_params=pltpu.CompilerParams(use_tc_tiling_on_sc=False),
    )
    return deg, scatter


def _scale_body(degp_ref, x_ref, p4_ref, dinv_ref):
    ones = jnp.ones((NW, 1), jnp.float32)
    deg = lax.dot_general(
        degp_ref[...], ones, (((0,), (0,)), ((), ())),
        preferred_element_type=jnp.float32,
    )
    dinv = lax.rsqrt(deg + 1.0)
    dinv_ref[...] = dinv
    xd = x_ref[...] * dinv[:N]
    for q in range(4):
        p4_ref[q, :N] = xd[:, q * QD:(q + 1) * QD]
        p4_ref[q, N:] = jnp.zeros((P_R - N, QD), jnp.float32)


def _final_body(acc_ref, x_ref, dinv_ref, w_ref, b_ref, a_ref, o_ref):
    ssum = jnp.concatenate([acc_ref[q, :N] for q in range(4)], axis=1)
    dinv = dinv_ref[:N]
    q = (ssum + x_ref[...] * dinv) * dinv
    z = jnp.dot(q, w_ref[...], preferred_element_type=jnp.float32) + b_ref[...]
    o_ref[...] = jnp.where(z >= 0, z, a_ref[...] * z)


def kernel(x, edge_index, W, b, prelu_a):
    pad = jnp.full((E_PAD - E,), N, jnp.int32)
    src3d = jnp.concatenate([edge_index[0], pad]).reshape(
        NS, NGROUP * GROUP, CHUNK)
    dst3d = jnp.concatenate([edge_index[1], pad]).reshape(
        NS, NGROUP * GROUP, CHUNK)
    zrows = jnp.zeros((STRIPE, QD), jnp.float32)

    deg_kernel, scatter_kernel = _sc_kernels()
    degp = deg_kernel(edge_index[1])
    p4, dinv = pl.pallas_call(
        _scale_body,
        out_shape=(
            jax.ShapeDtypeStruct((4, P_R, QD), jnp.float32),
            jax.ShapeDtypeStruct((ACC_R, 1), jnp.float32),
        ),
    )(degp, x)
    acc4 = scatter_kernel(p4, src3d, dst3d, zrows)
    out = pl.pallas_call(
        _final_body,
        out_shape=jax.ShapeDtypeStruct((N, D), jnp.float32),
    )(acc4, x, dinv, W, b.reshape(1, D), prelu_a.reshape(1, 1))
    return out


# speedup vs baseline: 1.3495x; 1.0085x over previous
import functools

import jax
import jax.numpy as jnp
from jax import lax
from jax.experimental import pallas as pl
from jax.experimental.pallas import tpu as pltpu
from jax.experimental.pallas import tpu_sc as plsc

N, E, D = 10000, 320000, 128
NC, NS, NW = 2, 16, 32
EPT = E // NW
QD = D // 4
CHUNK = 128
GROUP = 5
NGROUP = 32
E_PAD = NS * NGROUP * GROUP * CHUNK
P_R = 10112
ACC_R = 10112
STRIPE = ACC_R // NS

def _deg_body(dst_hbm, degp_hbm, dchunk, deg_v):
    c = lax.axis_index("c")
    s = lax.axis_index("s")
    wid = s * NC + c
    pltpu.sync_copy(dst_hbm.at[pl.ds(wid * EPT, EPT)], dchunk)

    def zero(i, carry):
        deg_v[pl.ds(i * 16, 16)] = jnp.zeros((16,), jnp.float32)
        return carry

    lax.fori_loop(0, ACC_R // 16, zero, 0)

    ones = jnp.ones((16,), jnp.float32)

    def count(i, carry):
        idx = dchunk[pl.ds(i * 16, 16)]
        plsc.addupdate_scatter(deg_v, [idx], ones)
        return carry

    lax.fori_loop(0, EPT // 16, count, 0)
    pltpu.sync_copy(deg_v, degp_hbm.at[wid])


def _scatter_body(p4_hbm, src_hbm, dst_hbm, zrows_hbm, acc_hbm,
                  sidx_all, didx_all, rows_a, rows_b, acc_sh, p_sh,
                  gsem, ssem):
    c = lax.axis_index("c")
    s = lax.axis_index("s")
    row0 = s * STRIPE

    pltpu.sync_copy(src_hbm.at[s], sidx_all)
    pltpu.sync_copy(dst_hbm.at[s], didx_all)

    def fire_gathers(g, rows):
        pltpu.async_copy(p_sh.at[sidx_all.at[g]], rows, gsem)

    def drain_gathers(g, rows):
        pltpu.make_async_copy(p_sh.at[sidx_all.at[g]], rows, gsem).wait()

    def scatters(g, rows):
        pltpu.async_copy(rows, acc_sh.at[didx_all.at[g]], ssem, add=True).wait()

    for t in range(2):
        qq = t * NC + c
        pltpu.sync_copy(p4_hbm.at[qq, pl.ds(row0, STRIPE)],
                        p_sh.at[pl.ds(row0, STRIPE)])
        pltpu.sync_copy(zrows_hbm, acc_sh.at[pl.ds(row0, STRIPE)])
        plsc.subcore_barrier()

        fire_gathers(0, rows_a)

        def body(t2, carry):
            g = t2 * 2
            fire_gathers(g + 1, rows_b)
            drain_gathers(g, rows_a)
            scatters(g, rows_a)

            @pl.when(t2 < NGROUP // 2 - 1)
            def _():
                fire_gathers(g + 2, rows_a)

            drain_gathers(g + 1, rows_b)
            scatters(g + 1, rows_b)
            return carry

        lax.fori_loop(0, NGROUP // 2, body, 0)
        plsc.subcore_barrier()
        pltpu.sync_copy(acc_sh.at[pl.ds(row0, STRIPE)],
                        acc_hbm.at[qq, pl.ds(row0, STRIPE)])


@functools.lru_cache(maxsize=None)
def _sc_kernels():
    mesh = plsc.VectorSubcoreMesh(core_axis_name="c", subcore_axis_name="s")
    deg = pl.kernel(
        _deg_body,
        mesh=mesh,
        out_type=jax.ShapeDtypeStruct((NW, ACC_R), jnp.float32),
        scratch_types=[
            pltpu.VMEM((EPT,), jnp.int32),
            pltpu.VMEM((ACC_R,), jnp.float32),
        ],
        compiler_params=pltpu.CompilerParams(needs_layout_passes=False),
    )
    scatter = pl.kernel(
        _scatter_body,
        mesh=mesh,
        out_type=jax.ShapeDtypeStruct((4, ACC_R, QD), jnp.float32),
        scratch_types=[
            pltpu.VMEM((NGROUP, GROUP * CHUNK), jnp.int32),
            pltpu.VMEM((NGROUP, GROUP * CHUNK), jnp.int32),
            pltpu.VMEM((GROUP * CHUNK, QD), jnp.float32),
            pltpu.VMEM((GROUP * CHUNK, QD), jnp.float32),
            pltpu.VMEM_SHARED((ACC_R, QD), jnp.float32),
            pltpu.VMEM_SHARED((P_R, QD), jnp.float32),
            pltpu.SemaphoreType.DMA,
            pltpu.SemaphoreType.DMA,
        ],
        compiler_params=pltpu.CompilerParams(use_tc_tiling_on_sc=False),
    )
    return deg, scatter


def _scale_body(degp_ref, x_ref, p4_ref, dinv_ref):
    ones = jnp.ones((NW, 1), jnp.float32)
    deg = lax.dot_general(
        degp_ref[...], ones, (((0,), (0,)), ((), ())),
        preferred_element_type=jnp.float32,
    )
    dinv = lax.rsqrt(deg + 1.0)
    dinv_ref[...] = dinv
    xd = x_ref[...] * dinv[:N]
    for q in range(4):
        p4_ref[q, :N] = xd[:, q * QD:(q + 1) * QD]
        p4_ref[q, N:] = jnp.zeros((P_R - N, QD), jnp.float32)


def _final_body(acc_ref, x_ref, dinv_ref, w_ref, b_ref, a_ref, o_ref):
    ssum = jnp.concatenate([acc_ref[q, :N] for q in range(4)], axis=1)
    dinv = dinv_ref[:N]
    q = (ssum + x_ref[...] * dinv) * dinv
    z = jnp.dot(q, w_ref[...], preferred_element_type=jnp.float32) + b_ref[...]
    o_ref[...] = jnp.where(z >= 0, z, a_ref[...] * z)


def kernel(x, edge_index, W, b, prelu_a):
    pad = jnp.full((E_PAD - E,), N, jnp.int32)
    src3d = jnp.concatenate([edge_index[0], pad]).reshape(
        NS, NGROUP, GROUP * CHUNK)
    dst3d = jnp.concatenate([edge_index[1], pad]).reshape(
        NS, NGROUP, GROUP * CHUNK)
    zrows = jnp.zeros((STRIPE, QD), jnp.float32)

    deg_kernel, scatter_kernel = _sc_kernels()
    degp = deg_kernel(edge_index[1])
    p4, dinv = pl.pallas_call(
        _scale_body,
        out_shape=(
            jax.ShapeDtypeStruct((4, P_R, QD), jnp.float32),
            jax.ShapeDtypeStruct((ACC_R, 1), jnp.float32),
        ),
    )(degp, x)
    acc4 = scatter_kernel(p4, src3d, dst3d, zrows)
    out = pl.pallas_call(
        _final_body,
        out_shape=jax.ShapeDtypeStruct((N, D), jnp.float32),
    )(acc4, x, dinv, W, b.reshape(1, D), prelu_a.reshape(1, 1))
    return out
